# baseline (speedup 1.0000x reference)
import functools

import jax
import jax.numpy as jnp
from jax import lax
from jax.experimental import pallas as pl
from jax.experimental.pallas import tpu as pltpu
from jax.experimental.pallas import tpu_sc as plsc

N_FIELDS = 26
VOCAB = 100000
DIM = 16
NC = 2
NS = 16
NW = NC * NS
G = 128
SUB = 8
CHUNK = SUB * G


def _gather_body(idx_hbm, table_hbm, out_hbm, idx_v, rows_v, sem0, sem1):
    nchunk = idx_hbm.shape[1]
    wid = lax.axis_index("s") * NC + lax.axis_index("c")

    pltpu.sync_copy(idx_hbm.at[wid], idx_v)

    def fire(j, s, sem):
        for k in range(SUB):
            pltpu.async_copy(
                table_hbm.at[idx_v.at[j, k]], rows_v.at[s, k], sem
            )

    def drain(j, s, sem):
        for k in range(SUB):
            pltpu.make_async_copy(
                table_hbm.at[idx_v.at[j, k]], rows_v.at[s, k], sem
            ).wait()

    fire(0, 0, sem0)

    def step(j, s, sem_cur, sem_nxt):
        @pl.when(j + 1 < nchunk)
        def _():
            fire(j + 1, 1 - s, sem_nxt)

        drain(j, s, sem_cur)
        pltpu.sync_copy(rows_v.at[s], out_hbm.at[wid].at[j])

    def body(j, carry):
        @pl.when(lax.rem(j, 2) == 0)
        def _():
            step(j, 0, sem0, sem1)

        @pl.when(lax.rem(j, 2) == 1)
        def _():
            step(j, 1, sem1, sem0)

        return carry

    lax.fori_loop(0, nchunk, body, 0, unroll=False)


def kernel(indices, tables):
    B = indices.shape[0]
    rows = B * N_FIELDS
    rw = rows // NW
    nchunk = rw // CHUNK

    flat_tables = tables.reshape(N_FIELDS * VOCAB, DIM)
    flat_idx = (
        indices + jnp.arange(N_FIELDS, dtype=jnp.int32) * VOCAB
    ).reshape(NW, nchunk, SUB, G)

    mesh = plsc.VectorSubcoreMesh(core_axis_name="c", subcore_axis_name="s")
    gather = functools.partial(
        pl.kernel,
        out_type=jax.ShapeDtypeStruct((NW, nchunk, SUB, G, DIM), jnp.float32),
        mesh=mesh,
        scratch_types=[
            pltpu.VMEM((nchunk, SUB, G), jnp.int32),
            pltpu.VMEM((2, SUB, G, DIM), jnp.float32),
            pltpu.SemaphoreType.DMA,
            pltpu.SemaphoreType.DMA,
        ],
        compiler_params=pltpu.CompilerParams(use_tc_tiling_on_sc=False),
    )(_gather_body)

    out = gather(flat_idx, flat_tables)
    return out.reshape(B, N_FIELDS * DIM)


# speedup vs baseline: 4.4237x; 4.4237x over previous
import functools

import jax
import jax.numpy as jnp
from jax import lax
from jax.experimental import pallas as pl
from jax.experimental.pallas import tpu as pltpu
from jax.experimental.pallas import tpu_sc as plsc

N_FIELDS = 26
VOCAB = 100000
DIM = 16
NC = 2
NS = 16
NW = NC * NS
TROWS = N_FIELDS * DIM
RPW = TROWS // NW
BCH = 4096


def _lookup_body(idx_hbm, tab_hbm, out_hbm, trow_v, idx_v, grow_v):
    batch = idx_hbm.shape[1]
    nch = batch // BCH
    wid = lax.axis_index("s") * NC + lax.axis_index("c")

    def row_loop(r, carry):
        row = wid * RPW + r
        f = row // DIM
        d = lax.rem(row, DIM)
        pltpu.sync_copy(tab_hbm.at[f, d], trow_v)

        def col_loop(c, carry2):
            pltpu.sync_copy(idx_hbm.at[f, pl.ds(c * BCH, BCH)], idx_v)

            def g(i, carry3):
                iv = idx_v[pl.ds(i * 16, 16)]
                grow_v[pl.ds(i * 16, 16)] = plsc.load_gather(trow_v, [iv])
                return carry3

            lax.fori_loop(0, BCH // 16, g, 0, unroll=4)
            pltpu.sync_copy(grow_v, out_hbm.at[row, pl.ds(c * BCH, BCH)])
            return carry2

        lax.fori_loop(0, nch, col_loop, 0)
        return carry

    lax.fori_loop(0, RPW, row_loop, 0)


def kernel(indices, tables):
    batch = indices.shape[0]

    idx_t = indices.T
    tab_t = jnp.transpose(tables, (0, 2, 1))

    mesh = plsc.VectorSubcoreMesh(core_axis_name="c", subcore_axis_name="s")
    lookup = functools.partial(
        pl.kernel,
        out_type=jax.ShapeDtypeStruct((TROWS, batch), jnp.float32),
        mesh=mesh,
        scratch_types=[
            pltpu.VMEM((VOCAB,), jnp.float32),
            pltpu.VMEM((BCH,), jnp.int32),
            pltpu.VMEM((BCH,), jnp.float32),
        ],
        compiler_params=pltpu.CompilerParams(
            use_tc_tiling_on_sc=True, needs_layout_passes=False
        ),
    )(_lookup_body)

    out_t = lookup(idx_t, tab_t)
    return out_t.T


# speedup vs baseline: 5.2644x; 1.1901x over previous
import functools

import jax
import jax.numpy as jnp
from jax import lax
from jax.experimental import pallas as pl
from jax.experimental.pallas import tpu as pltpu
from jax.experimental.pallas import tpu_sc as plsc

N_FIELDS = 26
VOCAB = 100000
DIM = 16
NC = 2
NS = 16
NW = NC * NS
TROWS = N_FIELDS * DIM
RPW = TROWS // NW
NSTR = 4
VCH = VOCAB // NSTR
BCH = 4096
NCH = 4


def _lookup_body(idx_hbm, tab_hbm, out_hbm, trow_v, idx_v, grow_v,
                 sem_t, sem_w0, sem_w1):
    wid = lax.axis_index("s") * NC + lax.axis_index("c")
    sem_w = (sem_w0, sem_w1)

    def row_loop(r, prev_f):
        row = wid * RPW + r
        f = row // DIM
        d = lax.rem(row, DIM)

        pltpu.async_copy(tab_hbm.at[f, d], trow_v, sem_t)

        @pl.when(f != prev_f)
        def _():
            pltpu.sync_copy(idx_hbm.at[f], idx_v)

        pltpu.make_async_copy(tab_hbm.at[f, d], trow_v, sem_t).wait()

        for c in range(NCH):
            s = c % 2
            if c >= 2:
                pltpu.make_async_copy(
                    grow_v.at[s], out_hbm.at[row, pl.ds(0, BCH)], sem_w[s]
                ).wait()
            else:
                @pl.when(r > 0)
                def _():
                    pltpu.make_async_copy(
                        grow_v.at[s], out_hbm.at[row, pl.ds(0, BCH)], sem_w[s]
                    ).wait()

            def g(i, carry3):
                iv = idx_v[pl.ds(c * BCH + i * 16, 16)]
                grow_v[s, pl.ds(i * 16, 16)] = plsc.load_gather(trow_v, [iv])
                return carry3

            lax.fori_loop(0, BCH // 16, g, 0, unroll=4)
            pltpu.async_copy(
                grow_v.at[s], out_hbm.at[row, pl.ds(c * BCH, BCH)], sem_w[s]
            )
        return f

    last_f = lax.fori_loop(0, RPW, row_loop, -1)
    last_row = wid * RPW + RPW - 1
    for s in range(2):
        pltpu.make_async_copy(
            grow_v.at[s], out_hbm.at[last_row, pl.ds(0, BCH)], sem_w[s]
        ).wait()


def kernel(indices, tables):
    batch = indices.shape[0]

    idx_t = indices.T
    tab_t = jnp.transpose(tables, (0, 2, 1))

    mesh = plsc.VectorSubcoreMesh(core_axis_name="c", subcore_axis_name="s")
    lookup = functools.partial(
        pl.kernel,
        out_type=jax.ShapeDtypeStruct((TROWS, batch), jnp.float32),
        mesh=mesh,
        scratch_types=[
            pltpu.VMEM((VOCAB,), jnp.float32),
            pltpu.VMEM((batch,), jnp.int32),
            pltpu.VMEM((2, BCH), jnp.float32),
            pltpu.SemaphoreType.DMA,
            pltpu.SemaphoreType.DMA,
            pltpu.SemaphoreType.DMA,
        ],
        compiler_params=pltpu.CompilerParams(
            use_tc_tiling_on_sc=True, needs_layout_passes=False
        ),
    )(_lookup_body)

    out_t = lookup(idx_t, tab_t)
    return out_t.T
